# baseline (speedup 1.0000x reference)
import functools

import jax
import jax.numpy as jnp
from jax import lax
from jax.experimental import pallas as pl
from jax.experimental.pallas import tpu as pltpu
from jax.experimental.pallas import tpu_sc as plsc

BYTES = 4
ED = 128
EDB = 32
B = 16384
NC, NS = 2, 16
NW = NC * NS
BPW = B // NW
CH = 128
TW = 1024 * (ED // 2)


def _pack_half(t):
    lo = lax.bitcast_convert_type(
        t[:, : ED // 2].astype(jnp.bfloat16), jnp.uint16
    ).astype(jnp.uint32)
    hi = lax.bitcast_convert_type(
        t[:, ED // 2:].astype(jnp.bfloat16), jnp.uint16
    ).astype(jnp.uint32)
    return (hi << 16) | lo


def _fuse_pack_body(bte_ref, bto_ref, w_ref, b_ref, o_ref):
    w = w_ref[...]
    for j in range(BYTES):
        wj = w[:, EDB * j:EDB * (j + 1)]
        for half, bt_ref in ((0, bte_ref), (1, bto_ref)):
            t = lax.dot_general(
                bt_ref[...], wj, (((1,), (1,)), ((), ())),
                preferred_element_type=jnp.float32,
            )
            if j == 0:
                t = t + b_ref[...]
            cs = half * (ED // 2)
            o_ref[pl.ds(j * 128, 128), pl.ds(cs, ED // 2)] = _pack_half(t)


def _fuse_pack(byte_table, W, b):
    return pl.pallas_call(
        _fuse_pack_body,
        out_shape=jax.ShapeDtypeStruct((512, ED), jnp.uint32),
    )(byte_table[0::2], byte_table[1::2], W, b.reshape(1, ED))


def _sc_embed(ints, tbl_packed):
    mesh = plsc.VectorSubcoreMesh(
        core_axis_name="c", subcore_axis_name="s", num_cores=NC, num_subcores=NS
    )

    @functools.partial(
        pl.kernel,
        out_type=jax.ShapeDtypeStruct((B, ED), jnp.float32),
        mesh=mesh,
        compiler_params=pltpu.CompilerParams(needs_layout_passes=False),
        scratch_types=[
            pltpu.VMEM((BPW,), jnp.int32),
            pltpu.VMEM((512, ED), jnp.uint32),
            pltpu.VMEM((CH, ED), jnp.float32),
            pltpu.VMEM((CH, ED), jnp.float32),
            pltpu.SemaphoreType.DMA,
            pltpu.SemaphoreType.DMA,
        ],
    )
    def body(ints_hbm, tbl_hbm, out_hbm, ints_v, tbl_v, out_a, out_b,
             sem_a, sem_b):
        wid = lax.axis_index("s") * NC + lax.axis_index("c")
        base = wid * BPW
        pltpu.sync_copy(tbl_hbm, tbl_v)
        pltpu.sync_copy(ints_hbm.at[pl.ds(base, BPW)], ints_v)
        mask_hi = jnp.uint32(0xFFFF0000)
        bufs = (out_a, out_b)
        sems = (sem_a, sem_b)

        def make_chunk(buf, off):
            def one(g, carry):
                v = ints_v[pl.ds(off + g * 16, 16)]
                for l in range(16):
                    s = v[l]
                    i = g * 16 + l
                    acc = [None] * (ED // 32)
                    for j in range(BYTES):
                        bj = (s >> (8 * (BYTES - 1 - j))) & 0xFF
                        m = (bj >> 1) + 128 * j
                        co = (bj & 1) << 6
                        for k in range(ED // 32):
                            u = plsc.bitcast(
                                tbl_v[m, pl.ds(co + 16 * k, 16)], jnp.bfloat16)
                            acc[k] = u if j == 0 else acc[k] + u
                    for k in range(ED // 32):
                        w = plsc.bitcast(acc[k], jnp.uint32)
                        buf[i, pl.ds(16 * k, 16)] = plsc.bitcast(
                            w << 16, jnp.float32)
                        buf[i, pl.ds(ED // 2 + 16 * k, 16)] = plsc.bitcast(
                            w & mask_hi, jnp.float32)
                return carry

            return one

        pending = [None, None]
        for c in range(BPW // CH):
            p = c % 2
            if pending[p] is not None:
                pending[p].wait()
            lax.fori_loop(0, CH // 16, make_chunk(bufs[p], c * CH), 0)
            pending[p] = pltpu.async_copy(
                bufs[p], out_hbm.at[pl.ds(base + c * CH, CH)], sems[p]
            )
        for p in range(2):
            if pending[p] is not None:
                pending[p].wait()

    return body(ints, tbl_packed)


def kernel(ints, byte_table, W, b):
    return _sc_embed(ints, _fuse_pack(byte_table, W, b))


# speedup vs baseline: 1.0422x; 1.0422x over previous
import functools

import jax
import jax.numpy as jnp
from jax import lax
from jax.experimental import pallas as pl
from jax.experimental.pallas import tpu as pltpu
from jax.experimental.pallas import tpu_sc as plsc

BYTES = 4
ED = 128
EDB = 32
B = 16384
NC, NS = 2, 16
NW = NC * NS
BPW = B // NW
CH = 128
TW = 1024 * (ED // 2)


def _pack_half(t):
    lo = lax.bitcast_convert_type(
        t[:, : ED // 2].astype(jnp.bfloat16), jnp.uint16
    ).astype(jnp.uint32)
    hi = lax.bitcast_convert_type(
        t[:, ED // 2:].astype(jnp.bfloat16), jnp.uint16
    ).astype(jnp.uint32)
    return (hi << 16) | lo


def _fuse_pack_body(bte_ref, bto_ref, w_ref, b_ref, o_ref):
    w = w_ref[...]
    for j in range(BYTES):
        wj = w[:, EDB * j:EDB * (j + 1)]
        for half, bt_ref in ((0, bte_ref), (1, bto_ref)):
            t = lax.dot_general(
                bt_ref[...], wj, (((1,), (1,)), ((), ())),
                preferred_element_type=jnp.float32,
            )
            if j == 0:
                t = t + b_ref[...]
            cs = half * (ED // 2)
            o_ref[pl.ds(j * 128, 128), pl.ds(cs, ED // 2)] = _pack_half(t)


def _fuse_pack(byte_table, W, b):
    return pl.pallas_call(
        _fuse_pack_body,
        out_shape=jax.ShapeDtypeStruct((512, ED), jnp.uint32),
    )(byte_table[0::2], byte_table[1::2], W, b.reshape(1, ED))


def _sc_embed(ints, tbl_packed):
    mesh = plsc.VectorSubcoreMesh(
        core_axis_name="c", subcore_axis_name="s", num_cores=NC, num_subcores=NS
    )

    @functools.partial(
        pl.kernel,
        out_type=jax.ShapeDtypeStruct((B, ED), jnp.float32),
        mesh=mesh,
        compiler_params=pltpu.CompilerParams(needs_layout_passes=False),
        scratch_types=[
            pltpu.VMEM((BPW,), jnp.int32),
            pltpu.VMEM((512, ED), jnp.uint32),
            pltpu.VMEM((CH, ED), jnp.float32),
            pltpu.VMEM((CH, ED), jnp.float32),
            pltpu.SemaphoreType.DMA,
            pltpu.SemaphoreType.DMA,
        ],
    )
    def body(ints_hbm, tbl_hbm, out_hbm, ints_v, tbl_v, out_a, out_b,
             sem_a, sem_b):
        wid = lax.axis_index("s") * NC + lax.axis_index("c")
        base = wid * BPW
        cp_t = pltpu.async_copy(tbl_hbm, tbl_v, sem_a)
        cp_i = pltpu.async_copy(ints_hbm.at[pl.ds(base, BPW)], ints_v, sem_b)
        cp_t.wait()
        cp_i.wait()
        mask_hi = jnp.uint32(0xFFFF0000)
        bufs = (out_a, out_b)
        sems = (sem_a, sem_b)

        def make_chunk(buf, off):
            def accumulate(s):
                acc = [None] * (ED // 32)
                for j in range(BYTES):
                    bj = (s >> (8 * (BYTES - 1 - j))) & 0xFF
                    m = (bj >> 1) + 128 * j
                    co = (bj & 1) << 6
                    for k in range(ED // 32):
                        u = plsc.bitcast(
                            tbl_v[m, pl.ds(co + 16 * k, 16)], jnp.bfloat16)
                        acc[k] = u if j == 0 else acc[k] + u
                return acc

            def store(i, acc):
                for k in range(ED // 32):
                    w = plsc.bitcast(acc[k], jnp.uint32)
                    buf[i, pl.ds(16 * k, 16)] = plsc.bitcast(
                        w << 16, jnp.float32)
                    buf[i, pl.ds(ED // 2 + 16 * k, 16)] = plsc.bitcast(
                        w & mask_hi, jnp.float32)

            def one(g, carry):
                v = ints_v[pl.ds(off + g * 16, 16)]
                for l in range(0, 16, 2):
                    s0, s1 = v[l], v[l + 1]
                    acc0 = accumulate(s0)
                    acc1 = accumulate(s1)
                    store(g * 16 + l, acc0)
                    store(g * 16 + l + 1, acc1)
                return carry

            return one

        pending = [None, None]
        for c in range(BPW // CH):
            p = c % 2
            if pending[p] is not None:
                pending[p].wait()
            lax.fori_loop(0, CH // 16, make_chunk(bufs[p], c * CH), 0)
            pending[p] = pltpu.async_copy(
                bufs[p], out_hbm.at[pl.ds(base + c * CH, CH)], sems[p]
            )
        for p in range(2):
            if pending[p] is not None:
                pending[p].wait()

    return body(ints, tbl_packed)


def kernel(ints, byte_table, W, b):
    return _sc_embed(ints, _fuse_pack(byte_table, W, b))


# speedup vs baseline: 1.1808x; 1.1331x over previous
import functools

import jax
import jax.numpy as jnp
from jax import lax
from jax.experimental import pallas as pl
from jax.experimental.pallas import tpu as pltpu
from jax.experimental.pallas import tpu_sc as plsc

BYTES = 4
ED = 128
EDB = 32
B = 16384
NC, NS = 2, 16
NW = NC * NS
BPW = B // NW
CH = 128
TW = 1024 * (ED // 2)


def _pack_half(t):
    lo = lax.bitcast_convert_type(
        t[:, : ED // 2].astype(jnp.bfloat16), jnp.uint16
    ).astype(jnp.uint32)
    hi = lax.bitcast_convert_type(
        t[:, ED // 2:].astype(jnp.bfloat16), jnp.uint16
    ).astype(jnp.uint32)
    return (hi << 16) | lo


def _fuse_pack_body(bte_ref, bto_ref, w_ref, b_ref, o_ref):
    w = w_ref[...]
    for j in range(BYTES):
        wj = w[:, EDB * j:EDB * (j + 1)]
        for half, bt_ref in ((0, bte_ref), (1, bto_ref)):
            t = lax.dot_general(
                bt_ref[...], wj, (((1,), (1,)), ((), ())),
                preferred_element_type=jnp.float32,
            )
            if j == 0:
                t = t + b_ref[...]
            cs = half * (ED // 2)
            o_ref[pl.ds(j * 128, 128), pl.ds(cs, ED // 2)] = _pack_half(t)


def _fuse_pack(byte_table, W, b):
    return pl.pallas_call(
        _fuse_pack_body,
        out_shape=jax.ShapeDtypeStruct((512, ED), jnp.uint32),
    )(byte_table[0::2], byte_table[1::2], W, b.reshape(1, ED))


def _sc_embed(ints, tbl_packed):
    mesh = plsc.VectorSubcoreMesh(
        core_axis_name="c", subcore_axis_name="s", num_cores=NC, num_subcores=NS
    )

    @functools.partial(
        pl.kernel,
        out_type=jax.ShapeDtypeStruct((B, ED), jnp.float32),
        mesh=mesh,
        compiler_params=pltpu.CompilerParams(needs_layout_passes=False),
        scratch_types=[
            pltpu.VMEM((BPW,), jnp.int32),
            pltpu.VMEM((512, ED), jnp.uint32),
            pltpu.VMEM_SHARED((512, ED), jnp.uint32),
            pltpu.VMEM((CH, ED), jnp.float32),
            pltpu.VMEM((CH, ED), jnp.float32),
            pltpu.SemaphoreType.DMA,
            pltpu.SemaphoreType.DMA,
        ],
    )
    def body(ints_hbm, tbl_hbm, out_hbm, ints_v, tbl_v, tbl_sh, out_a, out_b,
             sem_a, sem_b):
        sid = lax.axis_index("s")
        wid = sid * NC + lax.axis_index("c")
        base = wid * BPW
        cp_i = pltpu.async_copy(ints_hbm.at[pl.ds(base, BPW)], ints_v, sem_b)

        @pl.when(sid == 0)
        def _():
            pltpu.sync_copy(tbl_hbm, tbl_sh)

        plsc.subcore_barrier()
        cp_t = pltpu.async_copy(tbl_sh, tbl_v, sem_a)
        cp_t.wait()
        cp_i.wait()
        mask_hi = jnp.uint32(0xFFFF0000)
        bufs = (out_a, out_b)
        sems = (sem_a, sem_b)

        def make_chunk(buf, off):
            def accumulate(s):
                acc = [None] * (ED // 32)
                for j in range(BYTES):
                    bj = (s >> (8 * (BYTES - 1 - j))) & 0xFF
                    m = (bj >> 1) + 128 * j
                    co = (bj & 1) << 6
                    for k in range(ED // 32):
                        u = plsc.bitcast(
                            tbl_v[m, pl.ds(co + 16 * k, 16)], jnp.bfloat16)
                        acc[k] = u if j == 0 else acc[k] + u
                return acc

            def store(i, acc):
                for k in range(ED // 32):
                    w = plsc.bitcast(acc[k], jnp.uint32)
                    buf[i, pl.ds(16 * k, 16)] = plsc.bitcast(
                        w << 16, jnp.float32)
                    buf[i, pl.ds(ED // 2 + 16 * k, 16)] = plsc.bitcast(
                        w & mask_hi, jnp.float32)

            def one(g, carry):
                v = ints_v[pl.ds(off + g * 16, 16)]
                for l in range(0, 16, 2):
                    s0, s1 = v[l], v[l + 1]
                    acc0 = accumulate(s0)
                    acc1 = accumulate(s1)
                    store(g * 16 + l, acc0)
                    store(g * 16 + l + 1, acc1)
                return carry

            return one

        pending = [None, None]
        for c in range(BPW // CH):
            p = c % 2
            if pending[p] is not None:
                pending[p].wait()
            lax.fori_loop(0, CH // 16, make_chunk(bufs[p], c * CH), 0)
            pending[p] = pltpu.async_copy(
                bufs[p], out_hbm.at[pl.ds(base + c * CH, CH)], sems[p]
            )
        for p in range(2):
            if pending[p] is not None:
                pending[p].wait()

    return body(ints, tbl_packed)


def kernel(ints, byte_table, W, b):
    return _sc_embed(ints, _fuse_pack(byte_table, W, b))


# speedup vs baseline: 1.2562x; 1.0638x over previous
import functools

import jax
import jax.numpy as jnp
from jax import lax
from jax.experimental import pallas as pl
from jax.experimental.pallas import tpu as pltpu
from jax.experimental.pallas import tpu_sc as plsc

BYTES = 4
ED = 128
EDB = 32
B = 16384
NC, NS = 2, 16
NW = NC * NS
BPW = B // NW
CH = 128
TW = 1024 * (ED // 2)


def _pack_half(t):
    lo = lax.bitcast_convert_type(
        t[:, : ED // 2].astype(jnp.bfloat16), jnp.uint16
    ).astype(jnp.uint32)
    hi = lax.bitcast_convert_type(
        t[:, ED // 2:].astype(jnp.bfloat16), jnp.uint16
    ).astype(jnp.uint32)
    return (hi << 16) | lo


def _fuse_pack_body(bt_ref, w_ref, b_ref, o_ref):
    w = w_ref[...]
    bt = bt_ref[...]
    for j in range(BYTES):
        wj = w[:, EDB * j:EDB * (j + 1)]
        t = lax.dot_general(
            bt, wj, (((1,), (1,)), ((), ())),
            preferred_element_type=jnp.float32,
        )
        if j == 0:
            t = t + b_ref[...]
        o_ref[pl.ds((j & 1) * 256, 256),
              pl.ds((j >> 1) * (ED // 2), ED // 2)] = _pack_half(t)


def _fuse_pack(byte_table, W, b):
    return pl.pallas_call(
        _fuse_pack_body,
        out_shape=jax.ShapeDtypeStruct((512, ED), jnp.uint32),
    )(byte_table, W, b.reshape(1, ED))


def _sc_embed(ints, tbl_packed):
    mesh = plsc.VectorSubcoreMesh(
        core_axis_name="c", subcore_axis_name="s", num_cores=NC, num_subcores=NS
    )

    @functools.partial(
        pl.kernel,
        out_type=jax.ShapeDtypeStruct((B, ED), jnp.float32),
        mesh=mesh,
        compiler_params=pltpu.CompilerParams(needs_layout_passes=False),
        scratch_types=[
            pltpu.VMEM((BPW,), jnp.int32),
            pltpu.VMEM((512, ED), jnp.uint32),
            pltpu.VMEM_SHARED((512, ED), jnp.uint32),
            pltpu.VMEM((CH, ED), jnp.float32),
            pltpu.VMEM((CH, ED), jnp.float32),
            pltpu.SemaphoreType.DMA,
            pltpu.SemaphoreType.DMA,
        ],
    )
    def body(ints_hbm, tbl_hbm, out_hbm, ints_v, tbl_v, tbl_sh, out_a, out_b,
             sem_a, sem_b):
        sid = lax.axis_index("s")
        wid = sid * NC + lax.axis_index("c")
        base = wid * BPW
        cp_i = pltpu.async_copy(ints_hbm.at[pl.ds(base, BPW)], ints_v, sem_b)

        @pl.when(sid == 0)
        def _():
            pltpu.sync_copy(tbl_hbm, tbl_sh)

        plsc.subcore_barrier()
        cp_t = pltpu.async_copy(tbl_sh, tbl_v, sem_a)
        cp_t.wait()
        cp_i.wait()
        mask_hi = jnp.uint32(0xFFFF0000)
        bufs = (out_a, out_b)
        sems = (sem_a, sem_b)

        def make_chunk(buf, off):
            def accumulate(s):
                acc = [None] * (ED // 32)
                for j in range(BYTES):
                    bj = (s >> (8 * (BYTES - 1 - j))) & 0xFF
                    m = bj + (j & 1) * 256
                    co = (j >> 1) * (ED // 2)
                    for k in range(ED // 32):
                        u = plsc.bitcast(
                            tbl_v[m, pl.ds(co + 16 * k, 16)], jnp.bfloat16)
                        acc[k] = u if j == 0 else acc[k] + u
                return acc

            def store(i, acc):
                for k in range(ED // 32):
                    w = plsc.bitcast(acc[k], jnp.uint32)
                    buf[i, pl.ds(16 * k, 16)] = plsc.bitcast(
                        w << 16, jnp.float32)
                    buf[i, pl.ds(ED // 2 + 16 * k, 16)] = plsc.bitcast(
                        w & mask_hi, jnp.float32)

            def one(g, carry):
                v = ints_v[pl.ds(off + g * 16, 16)]
                for l in range(0, 16, 2):
                    s0, s1 = v[l], v[l + 1]
                    acc0 = accumulate(s0)
                    acc1 = accumulate(s1)
                    store(g * 16 + l, acc0)
                    store(g * 16 + l + 1, acc1)
                return carry

            return one

        pending = [None, None]
        for c in range(BPW // CH):
            p = c % 2
            if pending[p] is not None:
                pending[p].wait()
            lax.fori_loop(0, CH // 16, make_chunk(bufs[p], c * CH), 0)
            pending[p] = pltpu.async_copy(
                bufs[p], out_hbm.at[pl.ds(base + c * CH, CH)], sems[p]
            )
        for p in range(2):
            if pending[p] is not None:
                pending[p].wait()

    return body(ints, tbl_packed)


def kernel(ints, byte_table, W, b):
    return _sc_embed(ints, _fuse_pack(byte_table, W, b))
